# Initial kernel scaffold; baseline (speedup 1.0000x reference)
import jax
import jax.numpy as jnp
from jax.experimental import pallas as pl


def kernel(x, params):
    raise NotImplementedError("write your pallas kernel here")


# speedup vs baseline: 1.1929x; 1.1929x over previous
import functools

import numpy as np
import jax
import jax.numpy as jnp
from jax.experimental import pallas as pl
from jax.experimental.pallas import tpu as pltpu

_NEG_INF = float("-inf")
_HIGHEST = jax.lax.Precision.HIGHEST


def _mm_body(a_ref, b_ref, bias_ref, o_ref, acc_ref, *, nk, relu, b_nk, bf16,
             swap):
    k = pl.program_id(2)

    @pl.when(k == 0)
    def _init():
        acc_ref[...] = jnp.zeros_like(acc_ref)

    a = a_ref[...]
    b = b_ref[...]
    if bf16:
        a = a.astype(jnp.bfloat16)
        b = b.astype(jnp.bfloat16)
    if swap:
        acc_ref[...] += jax.lax.dot_general(
            b, a, (((0,), (1,)), ((), ())),
            preferred_element_type=jnp.float32).T
    else:
        if b_nk:
            dn = (((1,), (1,)), ((), ()))
        else:
            dn = (((1,), (0,)), ((), ()))
        acc_ref[...] += jax.lax.dot_general(
            a, b, dn, preferred_element_type=jnp.float32,
            precision=None if bf16 else _HIGHEST)

    @pl.when(k == nk - 1)
    def _fin():
        r = acc_ref[...] + bias_ref[0:1, :]
        if relu:
            r = jnp.maximum(r, 0.0)
        o_ref[...] = r


def _matmul(a, b, bias, *, relu, b_nk, bm, bn, bk, bf16=False, swap=False):
    m, kdim = a.shape
    n = b.shape[0] if b_nk else b.shape[1]
    assert m % bm == 0 and n % bn == 0 and kdim % bk == 0, (a.shape, b.shape)
    nm, nn, nk = m // bm, n // bn, kdim // bk
    bias8 = jnp.broadcast_to(bias.reshape(1, n), (8, n))
    if b_nk:
        b_spec = pl.BlockSpec((bn, bk), lambda i, j, k: (j, k))
    else:
        b_spec = pl.BlockSpec((bk, bn), lambda i, j, k: (k, j))
    return pl.pallas_call(
        functools.partial(_mm_body, nk=nk, relu=relu, b_nk=b_nk, bf16=bf16,
                          swap=swap),
        grid=(nm, nn, nk),
        in_specs=[
            pl.BlockSpec((bm, bk), lambda i, j, k: (i, k)),
            b_spec,
            pl.BlockSpec((8, bn), lambda i, j, k: (0, j)),
        ],
        out_specs=pl.BlockSpec((bm, bn), lambda i, j, k: (i, j)),
        out_shape=jax.ShapeDtypeStruct((m, n), jnp.float32),
        scratch_shapes=[pltpu.VMEM((bm, bn), jnp.float32)],
        compiler_params=pltpu.CompilerParams(
            dimension_semantics=("parallel", "parallel", "arbitrary")),
    )(a, b, bias8)


def _pad_rows(x, mult):
    m = x.shape[0]
    pm = (-m) % mult
    if pm:
        x = jnp.pad(x, ((0, pm),) + ((0, 0),) * (x.ndim - 1))
    return x


def _conv3x3_hwc(x, w, b, stride, bm, swap=False):
    h, wd, c = x.shape
    o = w.shape[0]
    oh = -(-h // stride)
    ow = -(-wd // stride)
    pad_h = max((oh - 1) * stride + 3 - h, 0)
    pad_w = max((ow - 1) * stride + 3 - wd, 0)
    xp = jnp.pad(x, ((pad_h // 2, pad_h - pad_h // 2),
                     (pad_w // 2, pad_w - pad_w // 2), (0, 0)))
    taps = []
    for dy in range(3):
        for dx in range(3):
            taps.append(jax.lax.slice(
                xp, (dy, dx, 0),
                (dy + stride * (oh - 1) + 1, dx + stride * (ow - 1) + 1, c),
                (stride, stride, 1)))
    cols = jnp.concatenate(taps, axis=-1).reshape(oh * ow, 9 * c)
    wm = w.transpose(2, 3, 1, 0).reshape(9 * c, o)
    cols = _pad_rows(cols, bm)
    out = _matmul(cols, wm, b, relu=True, b_nk=False,
                  bm=bm, bn=o, bk=9 * c, bf16=True, swap=swap)
    return out[:oh * ow].reshape(oh, ow, o)


def _nms_body(anc_ref, loc_ref, sco_ref, oy1_ref, ox1_ref, oy2_ref, ox2_ref,
              *, n_valid, pre_nms, post_nms, thresh, img_h, img_w, min_size):
    rows = anc_ref.shape[1]
    shp = (rows, 128)
    lin = (jax.lax.broadcasted_iota(jnp.int32, shp, 0) * 128
           + jax.lax.broadcasted_iota(jnp.int32, shp, 1))
    in_range = lin < n_valid

    ay1 = anc_ref[0, :, :]
    ax1 = anc_ref[1, :, :]
    ay2 = anc_ref[2, :, :]
    ax2 = anc_ref[3, :, :]
    dy = loc_ref[0, :, :]
    dx = loc_ref[1, :, :]
    dh = loc_ref[2, :, :]
    dw = loc_ref[3, :, :]
    s0 = sco_ref[0, :, :]
    s1 = sco_ref[1, :, :]

    ah = ay2 - ay1
    aw = ax2 - ax1
    cy = ay1 + 0.5 * ah
    cx = ax1 + 0.5 * aw
    ncy = dy * ah + cy
    ncx = dx * aw + cx
    nh = jnp.exp(dh) * ah
    nw = jnp.exp(dw) * aw
    py1 = jnp.clip(ncy - 0.5 * nh, 0.0, img_h)
    px1 = jnp.clip(ncx - 0.5 * nw, 0.0, img_w)
    py2 = jnp.clip(ncy + 0.5 * nh, 0.0, img_h)
    px2 = jnp.clip(ncx + 0.5 * nw, 0.0, img_w)

    mx = jnp.maximum(s0, s1)
    e0 = jnp.exp(s0 - mx)
    e1 = jnp.exp(s1 - mx)
    fg = e1 / (e0 + e1)

    valid = ((py2 - py1) >= min_size) & ((px2 - px1) >= min_size) & in_range
    fg = jnp.where(valid, fg, _NEG_INF)

    ib = jax.lax.bitcast_convert_type(fg, jnp.int32)
    key = jnp.where(ib >= 0, ib, ib ^ jnp.int32(0x7FFFFFFF))

    def bs_body(_, carry):
        lo, hi = carry
        mid = (lo >> 1) + (hi >> 1) + (lo & hi & 1)
        cnt = jnp.sum((key >= mid).astype(jnp.int32))
        take = cnt >= pre_nms
        return jnp.where(take, mid, lo), jnp.where(take, hi, mid)

    lo, _ = jax.lax.fori_loop(
        0, 32, bs_body, (jnp.int32(-2**31), jnp.int32(2**31 - 1)))
    s_init = jnp.where(key >= lo, fg, _NEG_INF)

    area = (py2 - py1) * (px2 - px1)
    big = jnp.int32(2**31 - 1)
    m0 = jnp.max(fg)
    idx0 = jnp.min(jnp.where(fg == m0, lin, big))

    def body(i, s):
        m = jnp.max(s)
        exhausted = m == _NEG_INF
        idx = jnp.min(jnp.where(s == m, lin, big))
        idx = jnp.where(exhausted, idx0, idx)
        onehot = lin == idx
        by1 = jnp.sum(jnp.where(onehot, py1, 0.0))
        bx1 = jnp.sum(jnp.where(onehot, px1, 0.0))
        by2 = jnp.sum(jnp.where(onehot, py2, 0.0))
        bx2 = jnp.sum(jnp.where(onehot, px2, 0.0))
        ba = jnp.sum(jnp.where(onehot, area, 0.0))
        yy1 = jnp.maximum(by1, py1)
        xx1 = jnp.maximum(bx1, px1)
        yy2 = jnp.minimum(by2, py2)
        xx2 = jnp.minimum(bx2, px2)
        inter = jnp.maximum(yy2 - yy1, 0.0) * jnp.maximum(xx2 - xx1, 0.0)
        iou = inter / (ba + area - inter + 1e-9)
        s = jnp.where(iou > thresh, _NEG_INF, s)
        oy1_ref[pl.ds(i, 1), :] = by1.reshape(1, 1)
        ox1_ref[pl.ds(i, 1), :] = bx1.reshape(1, 1)
        oy2_ref[pl.ds(i, 1), :] = by2.reshape(1, 1)
        ox2_ref[pl.ds(i, 1), :] = bx2.reshape(1, 1)
        return s

    jax.lax.fori_loop(0, post_nms, body, s_init)


@functools.lru_cache(maxsize=None)
def _anchor_planes(hf, wf, rows):
    ratios = np.array([0.5, 1.0, 2.0])
    scales = np.array([8.0, 16.0, 32.0])
    base = 16.0
    anc = []
    for r in ratios:
        for s in scales:
            hh = base * s * np.sqrt(r)
            ww = base * s * np.sqrt(1.0 / r)
            anc.append([-hh / 2.0, -ww / 2.0, hh / 2.0, ww / 2.0])
    base_anchors = np.array(anc, dtype=np.float32)
    sy, sx = np.meshgrid(np.arange(hf) * 16, np.arange(wf) * 16, indexing='ij')
    shifts = np.stack([sy.ravel(), sx.ravel(), sy.ravel(), sx.ravel()],
                      axis=1).astype(np.float32)
    anchors = (shifts[:, None, :] + base_anchors[None, :, :]).reshape(-1, 4)
    n = anchors.shape[0]
    planes = np.zeros((4, rows * 128), dtype=np.float32)
    planes[:, :n] = anchors.T
    return planes.reshape(4, rows, 128)


def _proposals_nms(locs, scos, hf, wf, img_h, img_w):
    n = locs.shape[0]
    rows = -(-n // 128)
    pad = rows * 128 - n

    def plane(v):
        return jnp.pad(v, ((0, pad),)).reshape(rows, 128)

    anc4 = jnp.asarray(_anchor_planes(hf, wf, rows))
    loc4 = jnp.stack([plane(locs[:, i]) for i in range(4)])
    sc2 = jnp.stack([plane(scos[:, i]) for i in range(2)])
    outs = pl.pallas_call(
        functools.partial(
            _nms_body, n_valid=n, pre_nms=6000, post_nms=300,
            thresh=0.7, img_h=float(img_h), img_w=float(img_w),
            min_size=16.0),
        out_shape=[jax.ShapeDtypeStruct((300, 1), jnp.float32)] * 4,
    )(anc4, loc4, sc2)
    return jnp.concatenate(outs, axis=1)


def _roi_body(rois_ref, feat_ref, o_ref, *, wf, hmax, wmax, roi_size):
    k = pl.program_id(0)
    i = k // roi_size
    j = k % roi_size
    fy1 = rois_ref[:, 0:1] * (1.0 / 16.0)
    fx1 = rois_ref[:, 1:2] * (1.0 / 16.0)
    fy2 = rois_ref[:, 2:3] * (1.0 / 16.0)
    fx2 = rois_ref[:, 3:4] * (1.0 / 16.0)
    ty = (i.astype(jnp.float32) + 0.5) / roi_size
    tx = (j.astype(jnp.float32) + 0.5) / roi_size
    ys = fy1 + ty * (fy2 - fy1)
    xs = fx1 + tx * (fx2 - fx1)
    y0 = jnp.clip(jnp.floor(ys).astype(jnp.int32), 0, hmax)
    x0 = jnp.clip(jnp.floor(xs).astype(jnp.int32), 0, wmax)
    y1 = jnp.clip(y0 + 1, 0, hmax)
    x1 = jnp.clip(x0 + 1, 0, wmax)
    wy = jnp.clip(ys - y0.astype(jnp.float32), 0.0, 1.0)
    wx = jnp.clip(xs - x0.astype(jnp.float32), 0.0, 1.0)

    npos = feat_ref.shape[0]
    nroi = rois_ref.shape[0]
    niota = jax.lax.broadcasted_iota(jnp.int32, (nroi, npos), 1)
    t00 = y0 * wf + x0
    t01 = y0 * wf + x1
    t10 = y1 * wf + x0
    t11 = y1 * wf + x1
    w00 = (1.0 - wy) * (1.0 - wx)
    w01 = (1.0 - wy) * wx
    w10 = wy * (1.0 - wx)
    w11 = wy * wx
    sel = (jnp.where(niota == t00, w00, 0.0)
           + jnp.where(niota == t01, w01, 0.0)
           + jnp.where(niota == t10, w10, 0.0)
           + jnp.where(niota == t11, w11, 0.0))
    o_ref[0, :, :] = jax.lax.dot_general(
        sel, feat_ref[...], (((1,), (0,)), ((), ())),
        preferred_element_type=jnp.float32, precision=_HIGHEST)


def _roi_align_pooled(rois, feat_hw_c, hf, wf, roi_size):
    npos_pad = _pad_rows(feat_hw_c, 8).shape[0]
    featp = _pad_rows(feat_hw_c, 8)
    c = featp.shape[1]
    nroi = rois.shape[0]
    nbin = roi_size * roi_size
    out = pl.pallas_call(
        functools.partial(_roi_body, wf=wf, hmax=hf - 1, wmax=wf - 1,
                          roi_size=roi_size),
        grid=(nbin,),
        in_specs=[
            pl.BlockSpec((nroi, 4), lambda k: (0, 0)),
            pl.BlockSpec((npos_pad, c), lambda k: (0, 0)),
        ],
        out_specs=pl.BlockSpec((1, nroi, c), lambda k: (k, 0, 0)),
        out_shape=jax.ShapeDtypeStruct((nbin, nroi, c), jnp.float32),
        compiler_params=pltpu.CompilerParams(
            dimension_semantics=("arbitrary",)),
    )(rois, featp)
    return out.transpose(1, 2, 0).reshape(nroi, c * nbin)


def kernel(x, params):
    p = params
    img_h, img_w = x.shape[2], x.shape[3]
    xh = x[0].transpose(1, 2, 0)
    h = _conv3x3_hwc(xh, p['c1_w'], p['c1_b'], 2, bm=4096)
    h = _conv3x3_hwc(h, p['c2_w'], p['c2_b'], 2, bm=2048)
    h = _conv3x3_hwc(h, p['c3_w'], p['c3_b'], 2, bm=1280, swap=True)
    feat = _conv3x3_hwc(h, p['c4_w'], p['c4_b'], 2, bm=640, swap=True)
    hf, wf, cf = feat.shape
    r = _conv3x3_hwc(feat, p['rpn_conv_w'], p['rpn_conv_b'], 1, bm=640,
                     swap=True)

    r2 = _pad_rows(r.reshape(hf * wf, cf), 640)
    w_head = jnp.concatenate([p['rpn_loc_w'][:, :, 0, 0],
                              p['rpn_score_w'][:, :, 0, 0]], axis=0)
    b_head = jnp.concatenate([p['rpn_loc_b'], p['rpn_score_b']])
    w_head = jnp.pad(w_head, ((0, 64 - w_head.shape[0]), (0, 0)))
    b_head = jnp.pad(b_head, ((0, 64 - b_head.shape[0]),))
    rh = _matmul(r2, w_head, b_head, relu=False, b_nk=True,
                 bm=640, bn=64, bk=cf, bf16=True)[:hf * wf]
    locs = rh[:, :36].reshape(-1, 4)
    scos = rh[:, 36:54].reshape(-1, 2)

    rois = _proposals_nms(locs, scos, hf, wf, img_h, img_w)

    pooled = _roi_align_pooled(rois, feat.reshape(hf * wf, cf), hf, wf, 7)

    a = _pad_rows(pooled, 304)
    a = _matmul(a, p['fc1_w'], p['fc1_b'], relu=True, b_nk=True,
                bm=304, bn=512, bk=3584)
    a = _matmul(a, p['fc2_w'], p['fc2_b'], relu=True, b_nk=True,
                bm=304, bn=512, bk=4096)
    w_out = jnp.concatenate([p['cls_loc_w'], p['score_w']], axis=0)
    nout = w_out.shape[0]
    b_out = jnp.concatenate([p['cls_loc_b'], p['score_b']])
    w_out = jnp.pad(w_out, ((0, 128 - nout), (0, 0)))
    b_out = jnp.pad(b_out, ((0, 128 - nout),))
    hh = _matmul(a, w_out, b_out, relu=False, b_nk=True,
                 bm=304, bn=128, bk=4096)
    roi_cls_locs = hh[:300, :84]
    roi_scores = hh[:300, 84:105]
    roi_indices = jnp.zeros((300,), jnp.int32)
    return roi_cls_locs, roi_scores, rois, roi_indices
